# Initial kernel scaffold; baseline (speedup 1.0000x reference)
import jax
import jax.numpy as jnp
from jax.experimental import pallas as pl


def kernel(inputs, w_gating, w1, w2, b1, b2):
    raise NotImplementedError("write your pallas kernel here")


# speedup vs baseline: 1.0085x; 1.0085x over previous
import functools
import math

import jax
import jax.numpy as jnp
from jax import lax
from jax.experimental import pallas as pl
from jax.experimental.pallas import tpu as pltpu
from jax.experimental.pallas import tpu_sc as plsc

DIM = 1024
HIDDEN = 4096
E = 8
B = 2
N = 2048
BN = B * N
EPS = 1e-9
CAP = max(min(N, int((N * 1.25) / E)), 4)
CAPF = float(CAP)
BC = B * CAP
NSLOT = E * BC
LOSS_SCALE = E * 1e-2 / (B * N * N)

TBLK = 512
NBB = N // TBLK
NB = BN // TBLK

SBLK = 1280
SCH = NSLOT // SBLK

HBLK = 512
NH = HIDDEN // HBLK

NC = 2
NS = 16
NW = NC * NS
RPW = NSLOT // NW
CH = 80
TPW = BN // NW
CHT = 32

_INV_SQRT2 = 1.0 / math.sqrt(2.0)


def _gating_body(x_ref, wg_ref, sd1_ref, sc1_ref, g1_ref, sd2_ref, sc2_ref,
                 g2_ref, loss_ref, cnt_ref):
    p = pl.program_id(0)
    i = pl.program_id(1)
    b = i // NBB
    first = (i % NBB) == 0
    last_in_batch = (i % NBB) == (NBB - 1)

    x = x_ref[...]
    logits = jnp.dot(x, wg_ref[...], preferred_element_type=jnp.float32)
    mx = jnp.max(logits, axis=-1, keepdims=True)
    ex = jnp.exp(logits - mx)
    raw = ex / jnp.sum(ex, axis=-1, keepdims=True)

    lane = lax.broadcasted_iota(jnp.int32, (TBLK, E), 1)
    g1 = jnp.max(raw, axis=-1, keepdims=True)
    idx1 = jnp.min(jnp.where(raw >= g1, lane, E), axis=-1, keepdims=True)
    mask1 = (lane == idx1).astype(jnp.float32)

    zero18 = jnp.zeros((1, E), jnp.float32)
    base1 = jnp.where(first, zero18, cnt_ref[0:1, :E])
    cnt1_new = base1 + jnp.sum(mask1, axis=0, keepdims=True)
    cnt_ref[0:1, :E] = cnt1_new

    rawsum = jnp.sum(raw, axis=0, keepdims=True)

    @pl.when(p == 0)
    def _():
        @pl.when(b == 0)
        def _():
            prev = jnp.where(first, zero18, cnt_ref[4:5, :E])
            cnt_ref[4:5, :E] = prev + rawsum

            @pl.when(last_in_batch)
            def _():
                cnt_ref[2:3, :E] = jnp.minimum(cnt1_new, CAPF)
                cnt_ref[6:7, :E] = cnt1_new

        @pl.when(b == 1)
        def _():
            prev = jnp.where(first, zero18, cnt_ref[5:6, :E])
            cnt_ref[5:6, :E] = prev + rawsum

            @pl.when(last_in_batch)
            def _():
                cnt_ref[3:4, :E] = jnp.minimum(cnt1_new, CAPF)
                cnt_ref[7:8, :E] = cnt1_new

    @pl.when(p == 1)
    def _():
        raw2 = raw * (1.0 - mask1)
        g2 = jnp.max(raw2, axis=-1, keepdims=True)
        idx2 = jnp.min(jnp.where(raw2 >= g2, lane, E), axis=-1, keepdims=True)
        mask2 = (lane == idx2).astype(jnp.float32)
        denom = g1 + g2 + EPS
        g1n = g1 / denom
        g2n = g2 / denom

        rr = lax.broadcasted_iota(jnp.int32, (TBLK, TBLK), 0)
        cc = lax.broadcasted_iota(jnp.int32, (TBLK, TBLK), 1)
        tri = (cc < rr).astype(jnp.float32)
        posb1 = jnp.dot(tri, mask1, preferred_element_type=jnp.float32)
        pos1 = posb1 + base1
        pos1_tok = jnp.sum(pos1 * mask1, axis=-1, keepdims=True)
        kept1 = pos1_tok < CAPF

        base2 = jnp.where(first, zero18, cnt_ref[1:2, :E])
        cnt_ref[1:2, :E] = base2 + jnp.sum(mask2, axis=0, keepdims=True)
        c1tot = jnp.where(b == 0, cnt_ref[2:3, :E], cnt_ref[3:4, :E])
        posb2 = jnp.dot(tri, mask2, preferred_element_type=jnp.float32)
        pos2 = posb2 + base2 + c1tot
        pos2_tok = jnp.sum(pos2 * mask2, axis=-1, keepdims=True)
        kept2 = pos2_tok < CAPF

        g1k = jnp.where(kept1, g1n, 0.0)
        g2k = jnp.where(kept2, g2n, 0.0)
        slot1 = idx1 * BC + b * CAP + pos1_tok.astype(jnp.int32)
        slot2 = idx2 * BC + b * CAP + pos2_tok.astype(jnp.int32)
        sd1_ref[...] = jnp.broadcast_to(jnp.where(kept1, slot1, NSLOT), (TBLK, E))
        sd2_ref[...] = jnp.broadcast_to(jnp.where(kept2, slot2, NSLOT), (TBLK, E))
        sc1_ref[...] = jnp.broadcast_to(jnp.where(kept1, slot1, 0), (TBLK, E))
        sc2_ref[...] = jnp.broadcast_to(jnp.where(kept2, slot2, 0), (TBLK, E))
        g1_ref[...] = jnp.broadcast_to(g1k, (TBLK, E))
        g2_ref[...] = jnp.broadcast_to(g2k, (TBLK, E))

        @pl.when((b == B - 1) & last_in_batch)
        def _():
            l0 = jnp.sum(cnt_ref[4:5, :E] * cnt_ref[6:7, :E]
                         + cnt_ref[5:6, :E] * cnt_ref[7:8, :E],
                         axis=(0, 1), keepdims=True)
            loss_ref[...] = l0 * LOSS_SCALE


def _gating(x2d, wg):
    io = jax.ShapeDtypeStruct((BN, E), jnp.int32)
    fo = jax.ShapeDtypeStruct((BN, E), jnp.float32)
    tok_spec = pl.BlockSpec((TBLK, E), lambda p, i: (i, 0))
    return pl.pallas_call(
        _gating_body,
        grid=(2, NB),
        in_specs=[
            pl.BlockSpec((TBLK, DIM), lambda p, i: (i, 0)),
            pl.BlockSpec((DIM, E), lambda p, i: (0, 0)),
        ],
        out_specs=[tok_spec, tok_spec, tok_spec, tok_spec, tok_spec, tok_spec,
                   pl.BlockSpec((1, 1), lambda p, i: (0, 0))],
        out_shape=[io, io, fo, io, io, fo,
                   jax.ShapeDtypeStruct((1, 1), jnp.float32)],
        scratch_shapes=[pltpu.VMEM((8, 128), jnp.float32)],
    )(x2d, wg)


def _table_body(sd1_ref, sd2_ref, out_ref):
    s = pl.program_id(0)
    t = pl.program_id(1)
    col1 = sd1_ref[:, 0:1]
    col2 = sd2_ref[:, 0:1]
    slot = lax.broadcasted_iota(jnp.int32, (TBLK, SBLK), 1) + s * SBLK
    vals = (lax.broadcasted_iota(jnp.int32, (TBLK, 1), 0)
            + (t * TBLK + 1)).astype(jnp.float32)
    oh = (col1 == slot).astype(jnp.float32) + (col2 == slot).astype(jnp.float32)
    part = jnp.sum(oh * vals, axis=0, keepdims=True)

    @pl.when(t == 0)
    def _():
        out_ref[...] = part

    @pl.when(t > 0)
    def _():
        out_ref[...] = out_ref[...] + part


def _table(sd1, sd2):
    return pl.pallas_call(
        _table_body,
        grid=(SCH, NB),
        in_specs=[
            pl.BlockSpec((TBLK, E), lambda s, t: (t, 0)),
            pl.BlockSpec((TBLK, E), lambda s, t: (t, 0)),
        ],
        out_specs=pl.BlockSpec((1, SBLK), lambda s, t: (0, s)),
        out_shape=jax.ShapeDtypeStruct((1, NSLOT), jnp.float32),
        compiler_params=pltpu.CompilerParams(
            dimension_semantics=("parallel", "arbitrary")),
    )(sd1, sd2)


def _sc_dispatch(table, x_pad):
    @functools.partial(
        pl.kernel,
        out_type=jax.ShapeDtypeStruct((NSLOT, DIM), jnp.float32),
        mesh=plsc.VectorSubcoreMesh(core_axis_name="c", subcore_axis_name="s"),
        scratch_types=[
            pltpu.VMEM((RPW,), jnp.int32),
            pltpu.VMEM((CH, DIM), jnp.float32),
            pltpu.SemaphoreType.DMA,
        ],
    )
    def k(table_hbm, xpad_hbm, out_hbm, idx_v, rows_v, sem):
        wid = lax.axis_index("s") * NC + lax.axis_index("c")
        base = wid * RPW
        pltpu.sync_copy(table_hbm.at[pl.ds(base, RPW)], idx_v)
        for j in range(RPW // CH):
            pltpu.async_copy(
                xpad_hbm.at[idx_v.at[pl.ds(j * CH, CH)]], rows_v, sem).wait()
            pltpu.sync_copy(rows_v, out_hbm.at[pl.ds(base + j * CH, CH)])

    return k(table, x_pad)


def _ffn_body(x_ref, w1_ref, b1_ref, w2_ref, b2_ref, out_ref):
    h = pl.program_id(1)
    x = x_ref[...]
    hp = jnp.dot(x, w1_ref[0], preferred_element_type=jnp.float32)
    hp = hp + b1_ref[...][None, :]
    act = 0.5 * hp * (1.0 + lax.erf(hp * _INV_SQRT2))
    part = jnp.dot(act, w2_ref[0], preferred_element_type=jnp.float32)

    @pl.when(h == 0)
    def _():
        out_ref[...] = part + b2_ref[...][None, :]

    @pl.when(h > 0)
    def _():
        out_ref[...] = out_ref[...] + part


def _ffn(disp, w1, w2, b1, b2):
    return pl.pallas_call(
        _ffn_body,
        grid=(E, NH),
        in_specs=[
            pl.BlockSpec((BC, DIM), lambda e, h: (e, 0)),
            pl.BlockSpec((1, DIM, HBLK), lambda e, h: (e, 0, h)),
            pl.BlockSpec((HBLK,), lambda e, h: (h,)),
            pl.BlockSpec((1, HBLK, DIM), lambda e, h: (e, h, 0)),
            pl.BlockSpec((DIM,), lambda e, h: (0,)),
        ],
        out_specs=pl.BlockSpec((BC, DIM), lambda e, h: (e, 0)),
        out_shape=jax.ShapeDtypeStruct((NSLOT, DIM), jnp.float32),
        compiler_params=pltpu.CompilerParams(
            dimension_semantics=("parallel", "arbitrary")),
    )(disp, w1, b1, w2, b2)


def _lane_splat(vec, idx):
    idxs = lax.broadcast_in_dim(idx, (16, 1), ())
    dn = lax.GatherDimensionNumbers(
        offset_dims=(), collapsed_slice_dims=(0,), start_index_map=(0,))
    return lax.gather(vec, idxs, dn, (1,),
                      mode=lax.GatherScatterMode.PROMISE_IN_BOUNDS)


def _sc_combine(s1, s2, g1, g2, eo):
    @functools.partial(
        pl.kernel,
        out_type=jax.ShapeDtypeStruct((BN, DIM), jnp.float32),
        mesh=plsc.VectorSubcoreMesh(core_axis_name="c", subcore_axis_name="s"),
        scratch_types=[
            pltpu.VMEM((TPW,), jnp.int32),
            pltpu.VMEM((TPW,), jnp.int32),
            pltpu.VMEM((TPW,), jnp.float32),
            pltpu.VMEM((TPW,), jnp.float32),
            pltpu.VMEM((CHT, DIM), jnp.float32),
            pltpu.VMEM((CHT, DIM), jnp.float32),
            pltpu.VMEM((CHT, DIM), jnp.float32),
            pltpu.SemaphoreType.DMA,
            pltpu.SemaphoreType.DMA,
        ],
    )
    def k(s1_hbm, s2_hbm, g1_hbm, g2_hbm, eo_hbm, out_hbm,
          s1v, s2v, g1v, g2v, r1, r2, ro, sem1, sem2):
        wid = lax.axis_index("s") * NC + lax.axis_index("c")
        base = wid * TPW
        pltpu.sync_copy(s1_hbm.at[pl.ds(base, TPW)], s1v)
        pltpu.sync_copy(s2_hbm.at[pl.ds(base, TPW)], s2v)
        pltpu.sync_copy(g1_hbm.at[pl.ds(base, TPW)], g1v)
        pltpu.sync_copy(g2_hbm.at[pl.ds(base, TPW)], g2v)
        for j in range(TPW // CHT):
            cp1 = pltpu.async_copy(
                eo_hbm.at[s1v.at[pl.ds(j * CHT, CHT)]], r1, sem1)
            cp2 = pltpu.async_copy(
                eo_hbm.at[s2v.at[pl.ds(j * CHT, CHT)]], r2, sem2)
            cp1.wait()
            cp2.wait()
            for tg in range(CHT // 16):
                gv1 = g1v[pl.ds(j * CHT + tg * 16, 16)]
                gv2 = g2v[pl.ds(j * CHT + tg * 16, 16)]

                def tbody(tl, carry, tg=tg, gv1=gv1, gv2=gv2):
                    sp1 = _lane_splat(gv1, tl)
                    sp2 = _lane_splat(gv2, tl)
                    tok = tg * 16 + tl

                    def vbody(v, c2):
                        off = pl.multiple_of(v * 16, 16)
                        a = r1[tok, pl.ds(off, 16)]
                        bb = r2[tok, pl.ds(off, 16)]
                        ro[tok, pl.ds(off, 16)] = sp1 * a + sp2 * bb
                        return c2

                    return lax.fori_loop(0, DIM // 16, vbody, carry)

                lax.fori_loop(0, 16, tbody, 0)
            pltpu.sync_copy(ro, out_hbm.at[pl.ds(base + j * CHT, CHT)])

    return k(s1, s2, g1, g2, eo)


def kernel(inputs, w_gating, w1, w2, b1, b2):
    x2d = inputs.reshape(BN, DIM)
    sd1, sc1, g1o, sd2, sc2, g2o, loss11 = _gating(x2d, w_gating)
    table = _table(sd1, sd2).reshape(NSLOT).astype(jnp.int32)
    x_pad = jnp.concatenate([jnp.zeros((1, DIM), jnp.float32), x2d], axis=0)
    disp = _sc_dispatch(table, x_pad)
    eo = _ffn(disp, w1, w2, b1, b2)
    out2d = _sc_combine(sc1[:, 0], sc2[:, 0], g1o[:, 0], g2o[:, 0], eo)
    return out2d.reshape(B, N, DIM), loss11[0, 0]


# speedup vs baseline: 1.0153x; 1.0068x over previous
import functools
import math

import jax
import jax.numpy as jnp
from jax import lax
from jax.experimental import pallas as pl
from jax.experimental.pallas import tpu as pltpu
from jax.experimental.pallas import tpu_sc as plsc

DIM = 1024
HIDDEN = 4096
E = 8
B = 2
N = 2048
BN = B * N
EPS = 1e-9
CAP = max(min(N, int((N * 1.25) / E)), 4)
CAPF = float(CAP)
BC = B * CAP
NSLOT = E * BC
LOSS_SCALE = E * 1e-2 / (B * N * N)

TBLK = 512
NBB = N // TBLK
NB = BN // TBLK

SBLK = 1280
SCH = NSLOT // SBLK

HBLK = 512
NH = HIDDEN // HBLK

NC = 2
NS = 16
NW = NC * NS
RPW = NSLOT // NW
CH = 80
TPW = BN // NW
CHT = 32

_INV_SQRT2 = 1.0 / math.sqrt(2.0)


def _gating_body(x_ref, wg_ref, sd1_ref, sc1_ref, g1_ref, sd2_ref, sc2_ref,
                 g2_ref, loss_ref, cnt_ref):
    p = pl.program_id(0)
    i = pl.program_id(1)
    b = i // NBB
    first = (i % NBB) == 0
    last_in_batch = (i % NBB) == (NBB - 1)

    x = x_ref[...]
    logits = jnp.dot(x, wg_ref[...], preferred_element_type=jnp.float32)
    mx = jnp.max(logits, axis=-1, keepdims=True)
    ex = jnp.exp(logits - mx)
    raw = ex / jnp.sum(ex, axis=-1, keepdims=True)

    lane = lax.broadcasted_iota(jnp.int32, (TBLK, E), 1)
    g1 = jnp.max(raw, axis=-1, keepdims=True)
    idx1 = jnp.min(jnp.where(raw >= g1, lane, E), axis=-1, keepdims=True)
    mask1 = (lane == idx1).astype(jnp.float32)

    zero18 = jnp.zeros((1, E), jnp.float32)
    base1 = jnp.where(first, zero18, cnt_ref[0:1, :E])
    cnt1_new = base1 + jnp.sum(mask1, axis=0, keepdims=True)
    cnt_ref[0:1, :E] = cnt1_new

    rawsum = jnp.sum(raw, axis=0, keepdims=True)

    @pl.when(p == 0)
    def _():
        @pl.when(b == 0)
        def _():
            prev = jnp.where(first, zero18, cnt_ref[4:5, :E])
            cnt_ref[4:5, :E] = prev + rawsum

            @pl.when(last_in_batch)
            def _():
                cnt_ref[2:3, :E] = jnp.minimum(cnt1_new, CAPF)
                cnt_ref[6:7, :E] = cnt1_new

        @pl.when(b == 1)
        def _():
            prev = jnp.where(first, zero18, cnt_ref[5:6, :E])
            cnt_ref[5:6, :E] = prev + rawsum

            @pl.when(last_in_batch)
            def _():
                cnt_ref[3:4, :E] = jnp.minimum(cnt1_new, CAPF)
                cnt_ref[7:8, :E] = cnt1_new

    @pl.when(p == 1)
    def _():
        raw2 = raw * (1.0 - mask1)
        g2 = jnp.max(raw2, axis=-1, keepdims=True)
        idx2 = jnp.min(jnp.where(raw2 >= g2, lane, E), axis=-1, keepdims=True)
        mask2 = (lane == idx2).astype(jnp.float32)
        denom = g1 + g2 + EPS
        g1n = g1 / denom
        g2n = g2 / denom

        rr = lax.broadcasted_iota(jnp.int32, (TBLK, TBLK), 0)
        cc = lax.broadcasted_iota(jnp.int32, (TBLK, TBLK), 1)
        tri = (cc < rr).astype(jnp.float32)
        posb1 = jnp.dot(tri, mask1, preferred_element_type=jnp.float32)
        pos1 = posb1 + base1
        pos1_tok = jnp.sum(pos1 * mask1, axis=-1, keepdims=True)
        kept1 = pos1_tok < CAPF

        base2 = jnp.where(first, zero18, cnt_ref[1:2, :E])
        cnt_ref[1:2, :E] = base2 + jnp.sum(mask2, axis=0, keepdims=True)
        c1tot = jnp.where(b == 0, cnt_ref[2:3, :E], cnt_ref[3:4, :E])
        posb2 = jnp.dot(tri, mask2, preferred_element_type=jnp.float32)
        pos2 = posb2 + base2 + c1tot
        pos2_tok = jnp.sum(pos2 * mask2, axis=-1, keepdims=True)
        kept2 = pos2_tok < CAPF

        g1k = jnp.where(kept1, g1n, 0.0)
        g2k = jnp.where(kept2, g2n, 0.0)
        slot1 = idx1 * BC + b * CAP + pos1_tok.astype(jnp.int32)
        slot2 = idx2 * BC + b * CAP + pos2_tok.astype(jnp.int32)
        sd1_ref[...] = jnp.broadcast_to(jnp.where(kept1, slot1, NSLOT), (TBLK, E))
        sd2_ref[...] = jnp.broadcast_to(jnp.where(kept2, slot2, NSLOT), (TBLK, E))
        sc1_ref[...] = jnp.broadcast_to(jnp.where(kept1, slot1, 0), (TBLK, E))
        sc2_ref[...] = jnp.broadcast_to(jnp.where(kept2, slot2, 0), (TBLK, E))
        g1_ref[...] = jnp.broadcast_to(g1k, (TBLK, E))
        g2_ref[...] = jnp.broadcast_to(g2k, (TBLK, E))

        @pl.when((b == B - 1) & last_in_batch)
        def _():
            l0 = jnp.sum(cnt_ref[4:5, :E] * cnt_ref[6:7, :E]
                         + cnt_ref[5:6, :E] * cnt_ref[7:8, :E],
                         axis=(0, 1), keepdims=True)
            loss_ref[...] = l0 * LOSS_SCALE


def _gating(x2d, wg):
    io = jax.ShapeDtypeStruct((BN, E), jnp.int32)
    fo = jax.ShapeDtypeStruct((BN, E), jnp.float32)
    tok_spec = pl.BlockSpec((TBLK, E), lambda p, i: (i, 0))
    return pl.pallas_call(
        _gating_body,
        grid=(2, NB),
        in_specs=[
            pl.BlockSpec((TBLK, DIM), lambda p, i: (i, 0)),
            pl.BlockSpec((DIM, E), lambda p, i: (0, 0)),
        ],
        out_specs=[tok_spec, tok_spec, tok_spec, tok_spec, tok_spec, tok_spec,
                   pl.BlockSpec((1, 1), lambda p, i: (0, 0))],
        out_shape=[io, io, fo, io, io, fo,
                   jax.ShapeDtypeStruct((1, 1), jnp.float32)],
        scratch_shapes=[pltpu.VMEM((8, 128), jnp.float32)],
    )(x2d, wg)


def _table_body(sd1_ref, sd2_ref, out_ref):
    s = pl.program_id(0)
    t = pl.program_id(1)
    col1 = sd1_ref[:, 0:1]
    col2 = sd2_ref[:, 0:1]
    slot = lax.broadcasted_iota(jnp.int32, (TBLK, SBLK), 1) + s * SBLK
    vals = (lax.broadcasted_iota(jnp.int32, (TBLK, 1), 0)
            + (t * TBLK + 1)).astype(jnp.float32)
    oh = (col1 == slot).astype(jnp.float32) + (col2 == slot).astype(jnp.float32)
    part = jnp.sum(oh * vals, axis=0, keepdims=True)

    @pl.when(t == 0)
    def _():
        out_ref[...] = part

    @pl.when(t > 0)
    def _():
        out_ref[...] = out_ref[...] + part


def _table(sd1, sd2):
    return pl.pallas_call(
        _table_body,
        grid=(SCH, NB),
        in_specs=[
            pl.BlockSpec((TBLK, E), lambda s, t: (t, 0)),
            pl.BlockSpec((TBLK, E), lambda s, t: (t, 0)),
        ],
        out_specs=pl.BlockSpec((1, SBLK), lambda s, t: (0, s)),
        out_shape=jax.ShapeDtypeStruct((1, NSLOT), jnp.float32),
        compiler_params=pltpu.CompilerParams(
            dimension_semantics=("parallel", "arbitrary")),
    )(sd1, sd2)


def _sc_dispatch(table, x_pad):
    @functools.partial(
        pl.kernel,
        out_type=jax.ShapeDtypeStruct((NSLOT, DIM), jnp.float32),
        mesh=plsc.VectorSubcoreMesh(core_axis_name="c", subcore_axis_name="s"),
        scratch_types=[
            pltpu.VMEM((RPW,), jnp.int32),
            pltpu.VMEM((CH, DIM), jnp.float32),
            pltpu.SemaphoreType.DMA,
        ],
    )
    def k(table_hbm, xpad_hbm, out_hbm, idx_v, rows_v, sem):
        wid = lax.axis_index("s") * NC + lax.axis_index("c")
        base = wid * RPW
        pltpu.sync_copy(table_hbm.at[pl.ds(base, RPW)], idx_v)
        for j in range(RPW // CH):
            pltpu.async_copy(
                xpad_hbm.at[idx_v.at[pl.ds(j * CH, CH)]], rows_v, sem).wait()
            pltpu.sync_copy(rows_v, out_hbm.at[pl.ds(base + j * CH, CH)])

    return k(table, x_pad)


def _ffn_body(x_ref, w1_ref, b1_ref, w2_ref, b2_ref, out_ref):
    h = pl.program_id(1)
    x = x_ref[...].astype(jnp.bfloat16)
    hp = jnp.dot(x, w1_ref[0].astype(jnp.bfloat16),
                 preferred_element_type=jnp.float32)
    hp = hp + b1_ref[...][None, :]
    act = 0.5 * hp * (1.0 + lax.erf(hp * _INV_SQRT2))
    part = jnp.dot(act.astype(jnp.bfloat16), w2_ref[0].astype(jnp.bfloat16),
                   preferred_element_type=jnp.float32)

    @pl.when(h == 0)
    def _():
        out_ref[...] = part + b2_ref[...][None, :]

    @pl.when(h > 0)
    def _():
        out_ref[...] = out_ref[...] + part


def _ffn(disp, w1, w2, b1, b2):
    return pl.pallas_call(
        _ffn_body,
        grid=(E, NH),
        in_specs=[
            pl.BlockSpec((BC, DIM), lambda e, h: (e, 0)),
            pl.BlockSpec((1, DIM, HBLK), lambda e, h: (e, 0, h)),
            pl.BlockSpec((HBLK,), lambda e, h: (h,)),
            pl.BlockSpec((1, HBLK, DIM), lambda e, h: (e, h, 0)),
            pl.BlockSpec((DIM,), lambda e, h: (0,)),
        ],
        out_specs=pl.BlockSpec((BC, DIM), lambda e, h: (e, 0)),
        out_shape=jax.ShapeDtypeStruct((NSLOT, DIM), jnp.float32),
        compiler_params=pltpu.CompilerParams(
            dimension_semantics=("parallel", "arbitrary")),
    )(disp, w1, b1, w2, b2)


def _lane_splat(vec, idx):
    idxs = lax.broadcast_in_dim(idx, (16, 1), ())
    dn = lax.GatherDimensionNumbers(
        offset_dims=(), collapsed_slice_dims=(0,), start_index_map=(0,))
    return lax.gather(vec, idxs, dn, (1,),
                      mode=lax.GatherScatterMode.PROMISE_IN_BOUNDS)


def _sc_combine(s1, s2, g1, g2, eo):
    @functools.partial(
        pl.kernel,
        out_type=jax.ShapeDtypeStruct((BN, DIM), jnp.float32),
        mesh=plsc.VectorSubcoreMesh(core_axis_name="c", subcore_axis_name="s"),
        scratch_types=[
            pltpu.VMEM((TPW,), jnp.int32),
            pltpu.VMEM((TPW,), jnp.int32),
            pltpu.VMEM((TPW,), jnp.float32),
            pltpu.VMEM((TPW,), jnp.float32),
            pltpu.VMEM((CHT, DIM), jnp.float32),
            pltpu.VMEM((CHT, DIM), jnp.float32),
            pltpu.VMEM((CHT, DIM), jnp.float32),
            pltpu.SemaphoreType.DMA,
            pltpu.SemaphoreType.DMA,
        ],
    )
    def k(s1_hbm, s2_hbm, g1_hbm, g2_hbm, eo_hbm, out_hbm,
          s1v, s2v, g1v, g2v, r1, r2, ro, sem1, sem2):
        wid = lax.axis_index("s") * NC + lax.axis_index("c")
        base = wid * TPW
        pltpu.sync_copy(s1_hbm.at[pl.ds(base, TPW)], s1v)
        pltpu.sync_copy(s2_hbm.at[pl.ds(base, TPW)], s2v)
        pltpu.sync_copy(g1_hbm.at[pl.ds(base, TPW)], g1v)
        pltpu.sync_copy(g2_hbm.at[pl.ds(base, TPW)], g2v)
        for j in range(TPW // CHT):
            cp1 = pltpu.async_copy(
                eo_hbm.at[s1v.at[pl.ds(j * CHT, CHT)]], r1, sem1)
            cp2 = pltpu.async_copy(
                eo_hbm.at[s2v.at[pl.ds(j * CHT, CHT)]], r2, sem2)
            cp1.wait()
            cp2.wait()
            for tg in range(CHT // 16):
                gv1 = g1v[pl.ds(j * CHT + tg * 16, 16)]
                gv2 = g2v[pl.ds(j * CHT + tg * 16, 16)]

                def tbody(tl, carry, tg=tg, gv1=gv1, gv2=gv2):
                    sp1 = _lane_splat(gv1, tl)
                    sp2 = _lane_splat(gv2, tl)
                    tok = tg * 16 + tl

                    @plsc.parallel_loop(0, DIM, step=16, unroll=8)
                    def _(off):
                        a = r1[tok, pl.ds(off, 16)]
                        bb = r2[tok, pl.ds(off, 16)]
                        ro[tok, pl.ds(off, 16)] = sp1 * a + sp2 * bb

                    return carry

                lax.fori_loop(0, 16, tbody, 0)
            pltpu.sync_copy(ro, out_hbm.at[pl.ds(base + j * CHT, CHT)])

    return k(s1, s2, g1, g2, eo)


def kernel(inputs, w_gating, w1, w2, b1, b2):
    x2d = inputs.reshape(BN, DIM)
    sd1, sc1, g1o, sd2, sc2, g2o, loss11 = _gating(x2d, w_gating)
    table = _table(sd1, sd2).reshape(NSLOT).astype(jnp.int32)
    x_pad = jnp.concatenate([jnp.zeros((1, DIM), jnp.float32), x2d], axis=0)
    disp = _sc_dispatch(table, x_pad)
    eo = _ffn(disp, w1, w2, b1, b2)
    out2d = _sc_combine(sc1[:, 0], sc2[:, 0], g1o[:, 0], g2o[:, 0], eo)
    return out2d.reshape(B, N, DIM), loss11[0, 0]
